# baseline (speedup 1.0000x reference)
import functools

import jax
import jax.numpy as jnp
import numpy as np
from jax import lax
from jax.experimental import pallas as pl
from jax.experimental.pallas import tpu as pltpu
from jax.experimental.pallas import tpu_sc as plsc

_B, _T = 32, 200
_VOCAB = 100000
_D = 128
_N_SLOTS = 8192
_K = 4
_GOLD = 0x9E3779B97F4A7C15

_P = 99
_PB = 112
_NCHUNK = _PB // 16
_QCOL = 120

_NC, _NS = 2, 16
_TILE_V = 25088
_NBLK = (_VOCAB + _TILE_V - 1) // _TILE_V

_I32 = np.int32


def _c32(v):
    return _I32(np.uint32(v & 0xFFFFFFFF).astype(np.int32))


def _srl(x, s):
    return lax.shift_right_logical(x, jnp.full((16,), s, jnp.int32))


def _carry(a, b, s):
    return _srl((a & b) | ((a | b) & ~s), 31)


def _mul64c(hi, lo, const):
    chi, clo = _c32(const >> 32), _c32(const)
    a0 = lo & _I32(0xFFFF)
    a1 = _srl(lo, 16)
    b0 = _I32(const & 0xFFFF)
    b1 = _I32((const >> 16) & 0xFFFF)
    p0 = a0 * b0
    p1a = a1 * b0
    p1b = a0 * b1
    t1 = p1a << _I32(16)
    lo1 = p0 + t1
    c1 = _carry(p0, t1, lo1)
    t2 = p1b << _I32(16)
    lo2 = lo1 + t2
    c2 = _carry(lo1, t2, lo2)
    hi_out = (a1 * b1 + _srl(p1a, 16) + _srl(p1b, 16) + c1 + c2
              + lo * chi + hi * clo)
    return hi_out, lo2


def _add64c(hi, lo, const):
    chi, clo = _c32(const >> 32), _c32(const)
    lo2 = lo + clo
    return hi + chi + _carry(lo, jnp.full((16,), clo, jnp.int32), lo2), lo2


def _xorshift(hi, lo, s):
    return hi ^ _srl(hi, s), lo ^ (_srl(lo, s) | (hi << _I32(32 - s)))


def _mix64_32(hi, lo):
    hi, lo = _xorshift(hi, lo, 30)
    hi, lo = _mul64c(hi, lo, 0xBF58476D1CE4E9B9)
    hi, lo = _xorshift(hi, lo, 27)
    hi, lo = _mul64c(hi, lo, 0x94D049BB133111EB)
    hi, lo = _xorshift(hi, lo, 31)
    return hi, lo


def _slots16(tok_i32):
    lo = tok_i32 ^ _I32(74565)
    hi = jnp.zeros((16,), jnp.int32)
    hhi, hlo = _mix64_32(hi, lo)
    out = []
    for k in range(_K):
        khi, klo = _add64c(hhi, hlo, (k * _GOLD) % (1 << 64))
        _, mlo = _mix64_32(khi, klo)
        out.append(mlo & _I32(_N_SLOTS - 1))
    return out


def _sc_reduce(karr, varr, table):
    mesh = plsc.VectorSubcoreMesh(core_axis_name="c", subcore_axis_name="s")

    @functools.partial(
        pl.kernel,
        mesh=mesh,
        out_type=jax.ShapeDtypeStruct((_B * _D,), jnp.float32),
        scratch_types=[
            pltpu.VMEM((_PB + 16,), jnp.int32),
            pltpu.VMEM((_PB,), jnp.int32),
            pltpu.VMEM((_PB, _D), jnp.float32),
            pltpu.VMEM((_D,), jnp.float32),
            pltpu.SemaphoreType.DMA,
        ],
    )
    def r_kernel(karr_hbm, varr_hbm, table_hbm, r_hbm, keys_v, idx_v,
                 rows_v, racc_v, sem):
        wid = lax.axis_index("s") * _NC + lax.axis_index("c")
        pltpu.sync_copy(varr_hbm.at[pl.ds(wid * _PB, _PB)], idx_v)
        gather = pltpu.async_copy(table_hbm.at[idx_v], rows_v, sem)
        pltpu.sync_copy(
            karr_hbm.at[pl.ds(wid * (_PB + 16), _PB + 16)], keys_v)

        iota = lax.iota(jnp.int32, 16)
        s31 = jnp.full((16,), 31, jnp.int32)

        def _isneg(v):
            return -lax.shift_right_arithmetic(v, s31)

        def _splat(vec, l):
            idx = jnp.full((16, 1), l, jnp.int32)
            return lax.gather(
                vec, idx,
                dimension_numbers=lax.GatherDimensionNumbers(
                    offset_dims=(), collapsed_slice_dims=(0,),
                    start_index_map=(0,)),
                slice_sizes=(1,),
                mode=lax.GatherScatterMode.PROMISE_IN_BOUNDS)

        qchunk = keys_v[pl.ds(_QCOL - _QCOL % 16, 16)]
        qtok = _splat(qchunk, _QCOL % 16)
        qslots = _slots16(qtok)

        c_chunks = []
        for j in range(_NCHUNK):
            kslots = _slots16(keys_v[pl.ds(16 * j, 16)])
            cnt = jnp.zeros((16,), jnp.int32)
            for k in range(_K):
                for kp in range(_K):
                    d = kslots[k] ^ qslots[kp]
                    cnt -= lax.shift_right_arithmetic(
                        d - _I32(1), jnp.full((16,), 31, jnp.int32))
            lane_valid = _isneg(iota + _I32(16 * j - _P)).astype(jnp.float32)
            c_chunks.append(
                cnt.astype(jnp.float32) * lane_valid * np.float32(1.0 / _K))

        gather.wait()

        accs = [jnp.zeros((16,), jnp.float32) for _ in range(_D // 16)]
        for p in range(_PB):
            w = _splat(c_chunks[p // 16], p % 16)
            for dj in range(_D // 16):
                row = rows_v[p, pl.ds(16 * dj, 16)]
                accs[dj] = accs[dj] + w * row
        for dj in range(_D // 16):
            racc_v[pl.ds(16 * dj, 16)] = accs[dj]
        pltpu.sync_copy(racc_v, r_hbm.at[pl.ds(wid * _D, _D)])

    return r_kernel(karr, varr, table)


def _i32(v):
    return jnp.asarray(v, dtype=jnp.int32)


def _mm_body(r_ref, w_ref, b_ref, out_ref):
    out_ref[...] = (
        jax.lax.dot_general(
            r_ref[...], w_ref[...],
            dimension_numbers=(((1,), (1,)), ((), ())),
            preferred_element_type=jnp.float32,
        )
        + b_ref[...]
    )


def _mm_call(r2d, w, b2d):
    return pl.pallas_call(
        _mm_body,
        grid=(_NBLK,),
        in_specs=[
            pl.BlockSpec((_B, _D), lambda i: (_i32(0), _i32(0))),
            pl.BlockSpec((_TILE_V, _D), lambda i: (i, _i32(0))),
            pl.BlockSpec((1, _TILE_V), lambda i: (_i32(0), i)),
        ],
        out_specs=pl.BlockSpec((_B, _TILE_V), lambda i: (_i32(0), i)),
        out_shape=jax.ShapeDtypeStruct((_B, _VOCAB), jnp.float32),
    )(r2d, w, b2d)


def kernel(x, emb_table, W, b):
    xi = x.astype(jnp.int32)
    keys = xi[:, 0:2 * _P:2]
    vals = xi[:, 1:2 * _P:2]
    q = xi[:, -1]
    zk = jnp.zeros((_B, _PB + 16 - _P), jnp.int32)
    karr = jnp.concatenate(
        [keys, zk.at[:, _QCOL - _P].set(q)], axis=1)
    varr = jnp.concatenate(
        [vals, jnp.zeros((_B, _PB - _P), jnp.int32)], axis=1)

    r = _sc_reduce(karr.reshape(-1), varr.reshape(-1), emb_table)
    return _mm_call(r.reshape(_B, _D), W, b.reshape(1, _VOCAB))


# speedup vs baseline: 1.2314x; 1.2314x over previous
import functools

import jax
import jax.numpy as jnp
import numpy as np
from jax import lax
from jax.experimental import pallas as pl
from jax.experimental.pallas import tpu as pltpu
from jax.experimental.pallas import tpu_sc as plsc

_B, _T = 32, 200
_VOCAB = 100000
_D = 128
_N_SLOTS = 8192
_K = 4
_SEED = np.uint64(74565)
_GOLD = np.uint64(0x9E3779B97F4A7C15)

_TS = np.arange(0, _T - 1, 2)
_TS = _TS[_TS + 1 < _T - 1]
_P = _TS.shape[0]
_PP = 104
_PH = 56

_NC, _NS = 2, 16
_NW = _NC * _NS
_TILE_V = 25088
_NBLK = (_VOCAB + _TILE_V - 1) // _TILE_V


def _mix64(x):
    x = x ^ (x >> np.uint64(30))
    x = x * np.uint64(0xBF58476D1CE4E9B9)
    x = x ^ (x >> np.uint64(27))
    x = x * np.uint64(0x94D049BB133111EB)
    x = x ^ (x >> np.uint64(31))
    return x


def _slots_of(tok):
    h = _mix64(tok ^ _SEED)
    probe = jnp.arange(_K, dtype=jnp.uint64)
    return (_mix64(h[..., None] + probe * _GOLD) % np.uint64(_N_SLOTS)).astype(
        jnp.int32)


def _sc_gather(table, idx):
    mesh = plsc.VectorSubcoreMesh(core_axis_name="c", subcore_axis_name="s")

    @functools.partial(
        pl.kernel,
        mesh=mesh,
        out_type=jax.ShapeDtypeStruct((_B * _PP, _D), jnp.float32),
        scratch_types=[
            pltpu.VMEM((_PH,), jnp.int32),
            pltpu.VMEM((_PP - _PH,), jnp.int32),
            pltpu.VMEM((_PH, _D), jnp.float32),
            pltpu.VMEM((_PP - _PH, _D), jnp.float32),
            pltpu.SemaphoreType.DMA,
            pltpu.SemaphoreType.DMA,
        ],
    )
    def gather_kernel(table_hbm, idx_hbm, out_hbm, idx_a, idx_b, rows_a,
                      rows_b, sem_a, sem_b):
        wid = lax.axis_index("s") * _NC + lax.axis_index("c")
        base = wid * _PP
        pltpu.sync_copy(idx_hbm.at[pl.ds(base, _PH)], idx_a)
        ga = pltpu.async_copy(table_hbm.at[idx_a], rows_a, sem_a)
        pltpu.sync_copy(idx_hbm.at[pl.ds(base + _PH, _PP - _PH)], idx_b)
        gb = pltpu.async_copy(table_hbm.at[idx_b], rows_b, sem_b)
        ga.wait()
        pltpu.sync_copy(rows_a, out_hbm.at[pl.ds(base, _PH)])
        gb.wait()
        pltpu.sync_copy(rows_b, out_hbm.at[pl.ds(base + _PH, _PP - _PH)])

    return gather_kernel(table, idx)


def _tc_body(slots_ref, qslots_ref, rows_ref, w_ref, b_ref, out_ref, r_ref):
    i = pl.program_id(0)

    @pl.when(i == 0)
    def _():
        q = qslots_ref[...]
        c = jnp.zeros((_B, _PP), jnp.float32)
        for k in range(_K):
            sk = slots_ref[k]
            for kp in range(_K):
                c += (sk == q[:, kp][:, None]).astype(jnp.float32)
        rows = rows_ref[...]
        r = jnp.sum(c[:, :, None] * rows, axis=1)
        r_ref[...] = r * (1.0 / _K)

    out_ref[...] = (
        jax.lax.dot_general(
            r_ref[...], w_ref[...],
            dimension_numbers=(((1,), (1,)), ((), ())),
            preferred_element_type=jnp.float32,
        )
        + b_ref[...]
    )


def _i32(v):
    return jnp.asarray(v, dtype=jnp.int32)


def _tc_call(slots_t, qslots, rows, w, b2d):
    return pl.pallas_call(
        _tc_body,
        grid=(_NBLK,),
        in_specs=[
            pl.BlockSpec((_K, _B, _PP), lambda i: (_i32(0), _i32(0), _i32(0))),
            pl.BlockSpec((_B, _K), lambda i: (_i32(0), _i32(0))),
            pl.BlockSpec((_B, _PP, _D),
                         lambda i: (_i32(0), _i32(0), _i32(0))),
            pl.BlockSpec((_TILE_V, _D), lambda i: (i, _i32(0))),
            pl.BlockSpec((1, _TILE_V), lambda i: (_i32(0), i)),
        ],
        out_specs=pl.BlockSpec((_B, _TILE_V), lambda i: (_i32(0), i)),
        out_shape=jax.ShapeDtypeStruct((_B, _VOCAB), jnp.float32),
        scratch_shapes=[pltpu.VMEM((_B, _D), jnp.float32)],
    )(slots_t, qslots, rows, w, b2d)


def kernel(x, emb_table, W, b):
    keys = x[:, _TS].astype(jnp.uint64)
    vals = x[:, _TS + 1].astype(jnp.int32)
    slots = _slots_of(keys)
    qslots = _slots_of(x[:, -1].astype(jnp.uint64))

    slots = jnp.concatenate(
        [slots, jnp.full((_B, _PP - _P, _K), -1, jnp.int32)], axis=1)
    slots_t = slots.transpose(2, 0, 1)
    vals = jnp.concatenate(
        [vals, jnp.zeros((_B, _PP - _P), jnp.int32)], axis=1)

    rows = _sc_gather(emb_table, vals.reshape(-1))
    rows = rows.reshape(_B, _PP, _D)

    return _tc_call(slots_t, qslots, rows, W, b.reshape(1, _VOCAB))
